# baseline (speedup 1.0000x reference)
import functools

import jax
import jax.numpy as jnp
from jax import lax
from jax.experimental import pallas as pl
from jax.experimental.pallas import tpu as pltpu
from jax.experimental.pallas import tpu_sc as plsc

B, T, V = 1024, 50, 1000
NC, NS = 2, 16
NW = NC * NS
VH = V // 8
XPAD = 1008
BH = B // 128

_mesh = plsc.VectorSubcoreMesh(core_axis_name="c", subcore_axis_name="s")


@functools.partial(
    pl.kernel,
    out_type=jax.ShapeDtypeStruct((T, VH, BH, 8, 128), jnp.float32),
    mesh=_mesh,
    scratch_types=[
        pltpu.VMEM((T, B), jnp.int32),
        pltpu.VMEM((8 * XPAD,), jnp.float32),
        pltpu.VMEM((BH, 8, 128), jnp.float32),
        pltpu.VMEM((BH, 8, 128), jnp.float32),
        pltpu.SemaphoreType.DMA,
        pltpu.SemaphoreType.DMA,
    ],
    compiler_params=pltpu.CompilerParams(
        use_tc_tiling_on_sc=False, needs_layout_passes=False
    ),
)
def _sc_gather(tableT_hbm, prevT_hbm, out_hbm, prevT_v, tile_v, buf_a, buf_b,
               sem_a, sem_b):
    wid = lax.axis_index("s") * NC + lax.axis_index("c")
    vh_lo = wid * VH // NW
    vh_hi = (wid + 1) * VH // NW
    pltpu.sync_copy(prevT_hbm, prevT_v)

    def fill(t, buf):
        @plsc.parallel_loop(0, BH, unroll=8)
        def _per_bh(bh):
            for blc in range(8):
                xs = prevT_v[t, pl.ds(bh * 128 + blc * 16, 16)]
                for vl in range(8):
                    g = plsc.load_gather(tile_v, [xs + vl * XPAD])
                    buf[bh, vl, pl.ds(blc * 16, 16)] = g

    def flush(t, vh, buf, sem):
        return pltpu.make_async_copy(buf, out_hbm.at[t, vh], sem)

    def per_vh(vh, carry):
        pltpu.sync_copy(tableT_hbm.at[pl.ds(vh * 8 * XPAD, 8 * XPAD)], tile_v)

        def per_pair(p, carry):
            first = jnp.logical_and(vh == vh_lo, p == 0)
            t = 2 * p

            @pl.when(jnp.logical_not(first))
            def _():
                flush(0, 0, buf_a, sem_a).wait()
            fill(t, buf_a)
            flush(t, vh, buf_a, sem_a).start()

            @pl.when(jnp.logical_not(first))
            def _():
                flush(0, 0, buf_b, sem_b).wait()
            fill(t + 1, buf_b)
            flush(t + 1, vh, buf_b, sem_b).start()
            return carry

        lax.fori_loop(0, T // 2, per_pair, 0)
        return carry

    lax.fori_loop(vh_lo, vh_hi, per_vh, 0)
    flush(0, 0, buf_a, sem_a).wait()
    flush(0, 0, buf_b, sem_b).wait()


def kernel(idx, log_probs):
    idx = idx.astype(jnp.int32)
    tableT = jnp.concatenate(
        [log_probs.T, jnp.zeros((V, XPAD - V), log_probs.dtype)], axis=1
    )
    prev = jnp.concatenate(
        [jnp.full((B, 1), V, jnp.int32), idx[:, :-1]], axis=1
    )
    out5 = _sc_gather(tableT.reshape(-1), prev.T)
    return out5.transpose(2, 4, 0, 1, 3).reshape(B, T, V)


# speedup vs baseline: 1.5161x; 1.5161x over previous
import functools

import jax
import jax.numpy as jnp
from jax import lax
from jax.experimental import pallas as pl
from jax.experimental.pallas import tpu as pltpu
from jax.experimental.pallas import tpu_sc as plsc

B, T, V = 1024, 50, 1000
NC, NS = 2, 16
NW = NC * NS
VH = V // 8
XPAD = 1008
BH = B // 128

_mesh = plsc.VectorSubcoreMesh(core_axis_name="c", subcore_axis_name="s")


@functools.partial(
    pl.kernel,
    out_type=jax.ShapeDtypeStruct((T, VH, BH, 8, 128), jnp.float32),
    mesh=_mesh,
    scratch_types=[
        pltpu.VMEM((T, B), jnp.int32),
        pltpu.VMEM((8 * XPAD,), jnp.float32),
        pltpu.VMEM((BH, 8, 128), jnp.float32),
        pltpu.VMEM((BH, 8, 128), jnp.float32),
        pltpu.SemaphoreType.DMA,
        pltpu.SemaphoreType.DMA,
    ],
    compiler_params=pltpu.CompilerParams(
        use_tc_tiling_on_sc=False, needs_layout_passes=False
    ),
)
def _sc_gather(tableT_hbm, prevT_hbm, out_hbm, prevT_v, tile_v, buf_a, buf_b,
               sem_a, sem_b):
    wid = lax.axis_index("s") * NC + lax.axis_index("c")
    vh_lo = wid * VH // NW
    vh_hi = (wid + 1) * VH // NW
    pltpu.sync_copy(prevT_hbm, prevT_v)

    def fill(t, buf):
        @plsc.parallel_loop(0, BH, unroll=4)
        def _per_bh(bh):
            for blc in range(8):
                xs = prevT_v[t, pl.ds(bh * 128 + blc * 16, 16)]
                for vl in range(8):
                    g = plsc.load_gather(tile_v, [xs + vl * XPAD])
                    buf[bh, vl, pl.ds(blc * 16, 16)] = g

    def flush(t, vh, buf, sem):
        return pltpu.make_async_copy(buf, out_hbm.at[t, vh], sem)

    def per_vh(vh, carry):
        pltpu.sync_copy(tableT_hbm.at[pl.ds(vh * 8 * XPAD, 8 * XPAD)], tile_v)

        def per_pair(p, carry):
            first = jnp.logical_and(vh == vh_lo, p == 0)
            t = 2 * p

            @pl.when(jnp.logical_not(first))
            def _():
                flush(0, 0, buf_a, sem_a).wait()
            fill(t, buf_a)
            flush(t, vh, buf_a, sem_a).start()

            @pl.when(jnp.logical_not(first))
            def _():
                flush(0, 0, buf_b, sem_b).wait()
            fill(t + 1, buf_b)
            flush(t + 1, vh, buf_b, sem_b).start()
            return carry

        lax.fori_loop(0, T // 2, per_pair, 0)
        return carry

    lax.fori_loop(vh_lo, vh_hi, per_vh, 0)
    flush(0, 0, buf_a, sem_a).wait()
    flush(0, 0, buf_b, sem_b).wait()


def kernel(idx, log_probs):
    idx = idx.astype(jnp.int32)
    tableT = jnp.concatenate(
        [log_probs.T, jnp.zeros((V, XPAD - V), log_probs.dtype)], axis=1
    )
    prev = jnp.concatenate(
        [jnp.full((B, 1), V, jnp.int32), idx[:, :-1]], axis=1
    )
    out5 = _sc_gather(tableT.reshape(-1), prev.T)
    return out5.transpose(2, 4, 0, 1, 3).reshape(B, T, V)


# speedup vs baseline: 2.5568x; 1.6864x over previous
import functools

import jax
import jax.numpy as jnp
from jax import lax
from jax.experimental import pallas as pl
from jax.experimental.pallas import tpu as pltpu
from jax.experimental.pallas import tpu_sc as plsc

B, T, V = 1024, 50, 1000
NC, NS = 2, 16
NW = NC * NS
VH = V // 8
XPAD = 1008
BH = B // 128

_mesh = plsc.VectorSubcoreMesh(core_axis_name="c", subcore_axis_name="s")


@functools.partial(
    pl.kernel,
    out_type=jax.ShapeDtypeStruct((T, VH, BH, 8, 128), jnp.float32),
    mesh=_mesh,
    scratch_types=[
        pltpu.VMEM((T, B), jnp.int32),
        pltpu.VMEM((8 * XPAD,), jnp.float32),
        pltpu.VMEM((BH, 8, 128), jnp.float32),
        pltpu.VMEM((BH, 8, 128), jnp.float32),
        pltpu.SemaphoreType.DMA,
        pltpu.SemaphoreType.DMA,
    ],
    compiler_params=pltpu.CompilerParams(
        use_tc_tiling_on_sc=False, needs_layout_passes=False
    ),
)
def _sc_gather(tableT_hbm, prevT_hbm, out_hbm, prevT_v, tile_v, buf_a, buf_b,
               sem_a, sem_b):
    wid = lax.axis_index("s") * NC + lax.axis_index("c")
    vh_lo = wid * VH // NW
    vh_hi = (wid + 1) * VH // NW
    pltpu.sync_copy(prevT_hbm, prevT_v)

    def fill(t, buf):
        @plsc.parallel_loop(0, BH * 8, unroll=8)
        def _per_chunk(c):
            bh = c // 8
            blc = c % 8
            xs = prevT_v[t, pl.ds(c * 16, 16)]
            for vl in range(8):
                g = plsc.load_gather(tile_v, [xs + vl * XPAD])
                buf[bh, vl, pl.ds(blc * 16, 16)] = g

    def flush(t, vh, buf, sem):
        return pltpu.make_async_copy(buf, out_hbm.at[t, vh], sem)

    def per_vh(vh, carry):
        pltpu.sync_copy(tableT_hbm.at[pl.ds(vh * 8 * XPAD, 8 * XPAD)], tile_v)

        def per_pair(p, carry):
            first = jnp.logical_and(vh == vh_lo, p == 0)
            t = 2 * p

            @pl.when(jnp.logical_not(first))
            def _():
                flush(0, 0, buf_a, sem_a).wait()
            fill(t, buf_a)
            flush(t, vh, buf_a, sem_a).start()

            @pl.when(jnp.logical_not(first))
            def _():
                flush(0, 0, buf_b, sem_b).wait()
            fill(t + 1, buf_b)
            flush(t + 1, vh, buf_b, sem_b).start()
            return carry

        lax.fori_loop(0, T // 2, per_pair, 0)
        return carry

    lax.fori_loop(vh_lo, vh_hi, per_vh, 0)
    flush(0, 0, buf_a, sem_a).wait()
    flush(0, 0, buf_b, sem_b).wait()


def kernel(idx, log_probs):
    idx = idx.astype(jnp.int32)
    tableT = jnp.concatenate(
        [log_probs.T, jnp.zeros((V, XPAD - V), log_probs.dtype)], axis=1
    )
    prev = jnp.concatenate(
        [jnp.full((B, 1), V, jnp.int32), idx[:, :-1]], axis=1
    )
    out5 = _sc_gather(tableT.reshape(-1), prev.T)
    return out5.transpose(2, 4, 0, 1, 3).reshape(B, T, V)
